# Initial kernel scaffold; baseline (speedup 1.0000x reference)
import jax
import jax.numpy as jnp
from jax.experimental import pallas as pl


def kernel(sentence, embedding, w_ih_t, w_hh_t, b_ih, b_hh, w_out_t, b_out):
    raise NotImplementedError("write your pallas kernel here")


# speedup vs baseline: 1.9077x; 1.9077x over previous
import functools

import jax
import jax.numpy as jnp
from jax.experimental import pallas as pl
from jax.experimental.pallas import tpu as pltpu


def _round_up(x, m):
    return -(-x // m) * m


def _gru_tagger_kernel(ids_ref, emb_hbm, wih_ref, whh_ref, bih_ref, bhh_ref,
                       wout_ref, bout_ref, out_ref, embeds_ref, hs_ref, sem,
                       *, L, E, HP):
    copies = []
    for t in range(L):
        c = pltpu.make_async_copy(
            emb_hbm.at[pl.ds(ids_ref[t], 1), :],
            embeds_ref.at[t],
            sem,
        )
        c.start()
        copies.append(c)
    for c in copies:
        c.wait()

    embeds = embeds_ref[...].reshape(L, E)
    gi_all = jnp.dot(embeds, wih_ref[...],
                     preferred_element_type=jnp.float32) + bih_ref[...]

    whh = whh_ref[...]
    bhh = bhh_ref[...]

    h = jnp.zeros((1, HP), jnp.float32)
    for t in range(L):
        gi = gi_all[t:t + 1, :]
        gh = jnp.dot(h, whh, preferred_element_type=jnp.float32) + bhh
        r = jax.nn.sigmoid(gi[:, 0 * HP:1 * HP] + gh[:, 0 * HP:1 * HP])
        z = jax.nn.sigmoid(gi[:, 1 * HP:2 * HP] + gh[:, 1 * HP:2 * HP])
        n = jnp.tanh(gi[:, 2 * HP:3 * HP] + r * gh[:, 2 * HP:3 * HP])
        h = (1.0 - z) * n + z * h
        hs_ref[t:t + 1, :] = h

    logits = jnp.dot(hs_ref[...], wout_ref[...],
                     preferred_element_type=jnp.float32) + bout_ref[...]
    m = jnp.max(logits, axis=-1, keepdims=True)
    shifted = logits - m
    lse = jnp.log(jnp.sum(jnp.exp(shifted), axis=-1, keepdims=True))
    out_ref[...] = shifted - lse


def _pad_gate_cols(w, H, HP):
    if HP == H:
        return w
    lead = w.shape[:-1]
    w3 = w.reshape(lead + (3, H))
    w3 = jnp.pad(w3, [(0, 0)] * (len(lead) + 1) + [(0, HP - H)])
    return w3.reshape(lead + (3 * HP,))


def kernel(sentence, embedding, w_ih_t, w_hh_t, b_ih, b_hh, w_out_t, b_out):
    L = sentence.shape[0]
    E = embedding.shape[1]
    H = w_hh_t.shape[0]
    T = w_out_t.shape[1]
    HP = _round_up(H, 128)

    w_ih_p = _pad_gate_cols(w_ih_t, H, HP)
    w_hh_p = _pad_gate_cols(w_hh_t, H, HP)
    if HP != H:
        w_hh_p = jnp.pad(w_hh_p, ((0, HP - H), (0, 0)))
    b_ih_p = _pad_gate_cols(b_ih, H, HP)
    b_hh_p = _pad_gate_cols(b_hh, H, HP)
    w_out_p = jnp.pad(w_out_t, ((0, HP - H), (0, 0))) if HP != H else w_out_t

    ids = sentence.astype(jnp.int32)

    kernel_fn = functools.partial(_gru_tagger_kernel, L=L, E=E, HP=HP)
    return pl.pallas_call(
        kernel_fn,
        out_shape=jax.ShapeDtypeStruct((L, T), jnp.float32),
        in_specs=[
            pl.BlockSpec(memory_space=pltpu.SMEM),
            pl.BlockSpec(memory_space=pl.ANY),
            pl.BlockSpec(memory_space=pltpu.VMEM),
            pl.BlockSpec(memory_space=pltpu.VMEM),
            pl.BlockSpec(memory_space=pltpu.VMEM),
            pl.BlockSpec(memory_space=pltpu.VMEM),
            pl.BlockSpec(memory_space=pltpu.VMEM),
            pl.BlockSpec(memory_space=pltpu.VMEM),
        ],
        out_specs=pl.BlockSpec(memory_space=pltpu.VMEM),
        scratch_shapes=[
            pltpu.VMEM((L, 1, E), jnp.float32),
            pltpu.VMEM((L, HP), jnp.float32),
            pltpu.SemaphoreType.DMA,
        ],
        compiler_params=pltpu.CompilerParams(
            disable_bounds_checks=True,
        ),
    )(ids, embedding, w_ih_p, w_hh_p, b_ih_p, b_hh_p, w_out_p, b_out)
